# baseline (speedup 1.0000x reference)
import functools

import jax
import jax.numpy as jnp
from jax import lax
from jax.experimental import pallas as pl
from jax.experimental.pallas import tpu as pltpu
from jax.experimental.pallas import tpu_sc as plsc

N_NODES = 10000
NODE_DIM = 256
EDGE_DIM = 16
HALF = 128
E_HALF = 160000
CHUNK = 256
SUB = 64
N_CHUNKS = E_HALF // CHUNK
NS = 16
LANES = 16
ZROWS = 8
N_PIECES = N_NODES // ZROWS
N_PITER = (N_PIECES + NS - 1) // NS
N_ITER = (N_CHUNKS + NS - 1) // NS


def _mm_body(x_ref, w_ref, o_ref):
    o_ref[0] = jnp.dot(x_ref[...], w_ref[...],
                       preferred_element_type=jnp.float32)


def _mm(x, w, mblk):
    m = x.shape[0]
    k = x.shape[1]
    return pl.pallas_call(
        _mm_body,
        out_shape=jax.ShapeDtypeStruct((2, m, HALF), jnp.float32),
        grid=(2, m // mblk),
        in_specs=[pl.BlockSpec((mblk, k), lambda j, i: (i, 0)),
                  pl.BlockSpec((k, HALF), lambda j, i: (0, j))],
        out_specs=pl.BlockSpec((1, mblk, HALF), lambda j, i: (j, i, 0)),
    )(x, w)


def _xe_body(a_ref, w_ref, b_ref, o_ref):
    o_ref[0] = (jnp.dot(a_ref[...], w_ref[...],
                        preferred_element_type=jnp.float32)
                + b_ref[pl.program_id(0)][None, :])


def _xe(attr, w1e, b1):
    mblk = 2000
    return pl.pallas_call(
        _xe_body,
        out_shape=jax.ShapeDtypeStruct((2, E_HALF, HALF), jnp.float32),
        grid=(2, E_HALF // mblk),
        in_specs=[pl.BlockSpec((mblk, EDGE_DIM), lambda j, i: (i, 0)),
                  pl.BlockSpec((EDGE_DIM, HALF), lambda j, i: (0, j)),
                  pl.BlockSpec((2, HALF), lambda j, i: (0, 0))],
        out_specs=pl.BlockSpec((1, mblk, HALF), lambda j, i: (j, i, 0)),
    )(attr, w1e, b1.reshape(2, HALF))


def _sc_main_body(tg, tr, xe, snd, rcv, s2_out,
                  idx_s, idx_r, idx_rg, bg, br, be, zrow, acc):
    c = lax.axis_index("c")
    wid = lax.axis_index("s")
    coff = c * N_NODES

    def _zfill(i, _):
        for k in range(HALF // LANES):
            zrow[i, pl.ds(k * LANES, LANES)] = jnp.zeros((LANES,),
                                                         jnp.float32)
        return 0
    lax.fori_loop(0, ZROWS, _zfill, 0)

    def _zero(i, _):
        p = wid + i * NS

        @pl.when(p < N_PIECES)
        def _():
            pltpu.sync_copy(zrow, acc.at[pl.ds(p * ZROWS, ZROWS)])
        return 0
    lax.fori_loop(0, N_PITER, _zero, 0)
    plsc.subcore_barrier()

    def _chunk(g):
        base = g * CHUNK
        pltpu.sync_copy(snd.at[g], idx_s)
        pltpu.sync_copy(rcv.at[g], idx_r)
        for j in range(CHUNK // SUB):
            for k in range(SUB // LANES):
                sl = pl.ds(k * LANES, LANES)
                idx_s[j, sl] = idx_s[j, sl] + coff
                idx_rg[j, sl] = idx_r[j, sl] + coff
        for j in range(CHUNK // SUB):
            pltpu.sync_copy(tg.at[idx_s.at[j]], bg)
            pltpu.sync_copy(tr.at[idx_rg.at[j]], br)
            pltpu.sync_copy(
                xe.at[pl.ds(c * E_HALF + base + j * SUB, SUB)], be)

            def _edge(e, _):
                for k in range(HALF // LANES):
                    sl = pl.ds(k * LANES, LANES)
                    x = bg[e, sl] + br[e, sl] + be[e, sl]
                    bg[e, sl] = x / (1.0 + jnp.exp(-x))
                return 0
            lax.fori_loop(0, SUB, _edge, 0, unroll=4)

            pltpu.sync_copy(bg, acc.at[idx_r.at[j]], add=True)

    def _outer(i, _):
        g = wid + i * NS

        @pl.when(g < N_CHUNKS)
        def _():
            _chunk(g)
        return 0
    lax.fori_loop(0, N_ITER, _outer, 0)
    plsc.subcore_barrier()

    def _drain(i, _):
        p = wid + i * NS

        @pl.when(p < N_PIECES)
        def _():
            r0 = p * ZROWS
            pltpu.sync_copy(acc.at[pl.ds(r0, ZROWS)],
                            s2_out.at[pl.ds(c * N_NODES + r0, ZROWS)])
        return 0
    lax.fori_loop(0, N_PITER, _drain, 0)


def _sc_cnt_body(rcv, cnt_out, idx_r, ones, zcnt, cntacc):
    c = lax.axis_index("c")
    wid = lax.axis_index("s")
    w = c * NS + wid

    def _zfill(i, _):
        for k in range(HALF // LANES):
            zcnt[i, pl.ds(k * LANES, LANES)] = jnp.zeros((LANES,),
                                                         jnp.float32)
        return 0
    lax.fori_loop(0, ZROWS, _zfill, 0)

    def _ofill(i, _):
        for k in range(HALF // LANES):
            ones[i, pl.ds(k * LANES, LANES)] = jnp.ones((LANES,),
                                                        jnp.float32)
        return 0
    lax.fori_loop(0, SUB, _ofill, 0)

    def _zero(i, _):
        p = wid + i * NS

        @pl.when(p < N_PIECES)
        def _():
            pltpu.sync_copy(zcnt, cntacc.at[pl.ds(p * ZROWS, ZROWS)])
        return 0
    lax.fori_loop(0, N_PITER, _zero, 0)
    plsc.subcore_barrier()

    n_witer = (N_CHUNKS + 2 * NS - 1) // (2 * NS)
    def _outer(i, _):
        g = w + i * 2 * NS

        @pl.when(g < N_CHUNKS)
        def _():
            pltpu.sync_copy(rcv.at[g], idx_r)
            for j in range(CHUNK // SUB):
                pltpu.sync_copy(ones, cntacc.at[idx_r.at[j]], add=True)
        return 0
    lax.fori_loop(0, n_witer, _outer, 0)
    plsc.subcore_barrier()

    def _drain(i, _):
        p = wid + i * NS

        @pl.when(p < N_PIECES)
        def _():
            r0 = p * ZROWS
            pltpu.sync_copy(cntacc.at[pl.ds(r0, ZROWS)],
                            cnt_out.at[pl.ds(c * N_NODES + r0, ZROWS)])
        return 0
    lax.fori_loop(0, N_PITER, _drain, 0)


@functools.lru_cache(maxsize=None)
def _sc_calls():
    mesh = plsc.VectorSubcoreMesh(core_axis_name="c", subcore_axis_name="s")
    main_call = functools.partial(
        pl.kernel,
        out_type=jax.ShapeDtypeStruct((2 * N_NODES, HALF), jnp.float32),
        mesh=mesh,
        scratch_types=[
            pltpu.VMEM((CHUNK // SUB, SUB), jnp.int32),
            pltpu.VMEM((CHUNK // SUB, SUB), jnp.int32),
            pltpu.VMEM((CHUNK // SUB, SUB), jnp.int32),
            pltpu.VMEM((SUB, HALF), jnp.float32),
            pltpu.VMEM((SUB, HALF), jnp.float32),
            pltpu.VMEM((SUB, HALF), jnp.float32),
            pltpu.VMEM((ZROWS, HALF), jnp.float32),
            pltpu.VMEM_SHARED((N_NODES, HALF), jnp.float32),
        ],
    )(_sc_main_body)
    cnt_call = functools.partial(
        pl.kernel,
        out_type=jax.ShapeDtypeStruct((2 * N_NODES, HALF), jnp.float32),
        mesh=mesh,
        scratch_types=[
            pltpu.VMEM((CHUNK // SUB, SUB), jnp.int32),
            pltpu.VMEM((SUB, HALF), jnp.float32),
            pltpu.VMEM((ZROWS, HALF), jnp.float32),
            pltpu.VMEM_SHARED((N_NODES, HALF), jnp.float32),
        ],
    )(_sc_cnt_body)
    return main_call, cnt_call


def _fin_body(s_ref, cnt_ref, hr_ref, w2_ref, b2_ref, g_ref, be_ref, o_ref):
    cnt = cnt_ref[0][:, 0:1] + cnt_ref[1][:, 0:1]
    msum = (jnp.dot(s_ref[0], w2_ref[:HALF],
                    preferred_element_type=jnp.float32)
            + jnp.dot(s_ref[1], w2_ref[HALF:],
                      preferred_element_type=jnp.float32)
            + cnt * b2_ref[...])
    x = hr_ref[...] + msum / jnp.maximum(cnt, 1.0)
    mu = jnp.mean(x, axis=-1, keepdims=True)
    var = jnp.mean((x - mu) ** 2, axis=-1, keepdims=True)
    o_ref[...] = (x - mu) * lax.rsqrt(var + 1e-5) * g_ref[...] + be_ref[...]


def _fin(s2, cnt2, h_regional, w2, b2, gamma, beta):
    mblk = 2000
    return pl.pallas_call(
        _fin_body,
        out_shape=jax.ShapeDtypeStruct((N_NODES, NODE_DIM), jnp.float32),
        grid=(N_NODES // mblk,),
        in_specs=[pl.BlockSpec((2, mblk, HALF), lambda i: (0, i, 0)),
                  pl.BlockSpec((2, mblk, HALF), lambda i: (0, i, 0)),
                  pl.BlockSpec((mblk, NODE_DIM), lambda i: (i, 0)),
                  pl.BlockSpec((NODE_DIM, NODE_DIM), lambda i: (0, 0)),
                  pl.BlockSpec((1, NODE_DIM), lambda i: (0, 0)),
                  pl.BlockSpec((1, NODE_DIM), lambda i: (0, 0)),
                  pl.BlockSpec((1, NODE_DIM), lambda i: (0, 0))],
        out_specs=pl.BlockSpec((mblk, NODE_DIM), lambda i: (i, 0)),
    )(s2, cnt2, h_regional, w2, b2.reshape(1, NODE_DIM),
      gamma.reshape(1, NODE_DIM), beta.reshape(1, NODE_DIM))


def kernel(h_global, h_regional, cross_edge_index, cross_edge_attr,
           n_global, W1, b1, W2, b2, gamma, beta):
    senders = cross_edge_index[0, :E_HALF].astype(jnp.int32)
    receivers = cross_edge_index[1, :E_HALF].astype(jnp.int32)
    attr = cross_edge_attr[:E_HALF]

    tg = _mm(h_global, W1[:NODE_DIM], 2000).reshape(2 * N_NODES, HALF)
    tr = _mm(h_regional, W1[NODE_DIM:2 * NODE_DIM], 2000).reshape(
        2 * N_NODES, HALF)
    xe = _xe(attr, W1[2 * NODE_DIM:], b1).reshape(2 * E_HALF, HALF)

    snd = senders.reshape(N_CHUNKS, CHUNK // SUB, SUB)
    rcv = receivers.reshape(N_CHUNKS, CHUNK // SUB, SUB)

    main_call, cnt_call = _sc_calls()
    s2 = main_call(tg, tr, xe, snd, rcv)
    cnt2 = cnt_call(rcv)
    return _fin(s2.reshape(2, N_NODES, HALF), cnt2.reshape(2, N_NODES, HALF),
                h_regional, W2, b2, gamma, beta)


# speedup vs baseline: 2.6815x; 2.6815x over previous
import functools

import jax
import jax.numpy as jnp
from jax import lax
from jax.experimental import pallas as pl
from jax.experimental.pallas import tpu as pltpu
from jax.experimental.pallas import tpu_sc as plsc

N_NODES = 10000
NODE_DIM = 256
EDGE_DIM = 16
HALF = 128
E_HALF = 160000
CHUNK = 256
SUB = 64
N_CHUNKS = E_HALF // CHUNK
NS = 16
LANES = 16
ZROWS = 8
N_PIECES = N_NODES // ZROWS
N_PITER = (N_PIECES + NS - 1) // NS
N_ITER = (N_CHUNKS + NS - 1) // NS


def _mm_body(x_ref, w_ref, o_ref):
    o_ref[0] = jnp.dot(x_ref[...], w_ref[...],
                       preferred_element_type=jnp.float32)


def _mm(x, w, mblk):
    m = x.shape[0]
    k = x.shape[1]
    return pl.pallas_call(
        _mm_body,
        out_shape=jax.ShapeDtypeStruct((2, m, HALF), jnp.float32),
        grid=(2, m // mblk),
        in_specs=[pl.BlockSpec((mblk, k), lambda j, i: (i, 0)),
                  pl.BlockSpec((k, HALF), lambda j, i: (0, j))],
        out_specs=pl.BlockSpec((1, mblk, HALF), lambda j, i: (j, i, 0)),
    )(x, w)


def _xe_body(a_ref, w_ref, b_ref, o_ref):
    o_ref[0] = (jnp.dot(a_ref[...], w_ref[...],
                        preferred_element_type=jnp.float32)
                + b_ref[pl.program_id(0)][None, :])


def _xe(attr, w1e, b1):
    mblk = 2000
    return pl.pallas_call(
        _xe_body,
        out_shape=jax.ShapeDtypeStruct((2, E_HALF, HALF), jnp.float32),
        grid=(2, E_HALF // mblk),
        in_specs=[pl.BlockSpec((mblk, EDGE_DIM), lambda j, i: (i, 0)),
                  pl.BlockSpec((EDGE_DIM, HALF), lambda j, i: (0, j)),
                  pl.BlockSpec((2, HALF), lambda j, i: (0, 0))],
        out_specs=pl.BlockSpec((1, mblk, HALF), lambda j, i: (j, i, 0)),
    )(attr, w1e, b1.reshape(2, HALF))


def _sc_main_body(tg, tr, xe, snd, rcv, s2_out,
                  idx_s, idx_r, idx_rg, bg0, bg1, br0, br1, be, zrow, acc,
                  sem_g):
    bgs = (bg0, bg1)
    brs = (br0, br1)
    c = lax.axis_index("c")
    wid = lax.axis_index("s")
    coff = c * N_NODES

    def _zfill(i, _):
        for k in range(HALF // LANES):
            zrow[i, pl.ds(k * LANES, LANES)] = jnp.zeros((LANES,),
                                                         jnp.float32)
        return 0
    lax.fori_loop(0, ZROWS, _zfill, 0)

    def _zero(i, _):
        p = wid + i * NS

        @pl.when(p < N_PIECES)
        def _():
            pltpu.sync_copy(zrow, acc.at[pl.ds(p * ZROWS, ZROWS)])
        return 0
    lax.fori_loop(0, N_PITER, _zero, 0)
    plsc.subcore_barrier()

    def _chunk(g):
        base = g * CHUNK
        pltpu.sync_copy(snd.at[g], idx_s)
        pltpu.sync_copy(rcv.at[g], idx_r)
        for j in range(CHUNK // SUB):
            for k in range(SUB // LANES):
                sl = pl.ds(k * LANES, LANES)
                idx_s[j, sl] = idx_s[j, sl] + coff
                idx_rg[j, sl] = idx_r[j, sl] + coff
        nd = [pltpu.async_copy(tg.at[idx_s.at[0]], bgs[0], sem_g),
              pltpu.async_copy(tr.at[idx_rg.at[0]], brs[0], sem_g)]
        for j in range(CHUNK // SUB):
            cur = j % 2
            bg, br = bgs[cur], brs[cur]
            for d in nd:
                d.wait()
            if j + 1 < CHUNK // SUB:
                nxt = 1 - cur
                nd = [pltpu.async_copy(tg.at[idx_s.at[j + 1]],
                                       bgs[nxt], sem_g),
                      pltpu.async_copy(tr.at[idx_rg.at[j + 1]],
                                       brs[nxt], sem_g)]
            pltpu.sync_copy(
                xe.at[pl.ds(c * E_HALF + base + j * SUB, SUB)], be)

            def _edge(e, _):
                for k in range(HALF // LANES):
                    sl = pl.ds(k * LANES, LANES)
                    x = bg[e, sl] + br[e, sl] + be[e, sl]
                    bg[e, sl] = x / (1.0 + jnp.exp(-x))
                return 0
            lax.fori_loop(0, SUB, _edge, 0)

            pltpu.sync_copy(bg, acc.at[idx_r.at[j]], add=True)

    def _outer(i, _):
        g = wid + i * NS

        @pl.when(g < N_CHUNKS)
        def _():
            _chunk(g)
        return 0
    lax.fori_loop(0, N_ITER, _outer, 0)
    plsc.subcore_barrier()

    def _drain(i, _):
        p = wid + i * NS

        @pl.when(p < N_PIECES)
        def _():
            r0 = p * ZROWS
            pltpu.sync_copy(acc.at[pl.ds(r0, ZROWS)],
                            s2_out.at[pl.ds(c * N_NODES + r0, ZROWS)])
        return 0
    lax.fori_loop(0, N_PITER, _drain, 0)


def _sc_cnt_body(rcv, cnt_out, idx_r, ones, zcnt, cntacc):
    c = lax.axis_index("c")
    wid = lax.axis_index("s")
    w = c * NS + wid

    def _zfill(i, _):
        for k in range(HALF // LANES):
            zcnt[i, pl.ds(k * LANES, LANES)] = jnp.zeros((LANES,),
                                                         jnp.float32)
        return 0
    lax.fori_loop(0, ZROWS, _zfill, 0)

    def _ofill(i, _):
        for k in range(HALF // LANES):
            ones[i, pl.ds(k * LANES, LANES)] = jnp.ones((LANES,),
                                                        jnp.float32)
        return 0
    lax.fori_loop(0, SUB, _ofill, 0)

    def _zero(i, _):
        p = wid + i * NS

        @pl.when(p < N_PIECES)
        def _():
            pltpu.sync_copy(zcnt, cntacc.at[pl.ds(p * ZROWS, ZROWS)])
        return 0
    lax.fori_loop(0, N_PITER, _zero, 0)
    plsc.subcore_barrier()

    n_witer = (N_CHUNKS + 2 * NS - 1) // (2 * NS)
    def _outer(i, _):
        g = w + i * 2 * NS

        @pl.when(g < N_CHUNKS)
        def _():
            pltpu.sync_copy(rcv.at[g], idx_r)
            for j in range(CHUNK // SUB):
                pltpu.sync_copy(ones, cntacc.at[idx_r.at[j]], add=True)
        return 0
    lax.fori_loop(0, n_witer, _outer, 0)
    plsc.subcore_barrier()

    def _drain(i, _):
        p = wid + i * NS

        @pl.when(p < N_PIECES)
        def _():
            r0 = p * ZROWS
            pltpu.sync_copy(cntacc.at[pl.ds(r0, ZROWS)],
                            cnt_out.at[pl.ds(c * N_NODES + r0, ZROWS)])
        return 0
    lax.fori_loop(0, N_PITER, _drain, 0)


@functools.lru_cache(maxsize=None)
def _sc_calls():
    mesh = plsc.VectorSubcoreMesh(core_axis_name="c", subcore_axis_name="s")
    main_call = functools.partial(
        pl.kernel,
        out_type=jax.ShapeDtypeStruct((2 * N_NODES, HALF), jnp.float32),
        mesh=mesh,
        scratch_types=[
            pltpu.VMEM((CHUNK // SUB, SUB), jnp.int32),
            pltpu.VMEM((CHUNK // SUB, SUB), jnp.int32),
            pltpu.VMEM((CHUNK // SUB, SUB), jnp.int32),
            pltpu.VMEM((SUB, HALF), jnp.float32),
            pltpu.VMEM((SUB, HALF), jnp.float32),
            pltpu.VMEM((SUB, HALF), jnp.float32),
            pltpu.VMEM((SUB, HALF), jnp.float32),
            pltpu.VMEM((SUB, HALF), jnp.float32),
            pltpu.VMEM((ZROWS, HALF), jnp.float32),
            pltpu.VMEM_SHARED((N_NODES, HALF), jnp.float32),
            pltpu.SemaphoreType.DMA,
        ],
    )(_sc_main_body)
    cnt_call = functools.partial(
        pl.kernel,
        out_type=jax.ShapeDtypeStruct((2 * N_NODES, HALF), jnp.float32),
        mesh=mesh,
        scratch_types=[
            pltpu.VMEM((CHUNK // SUB, SUB), jnp.int32),
            pltpu.VMEM((SUB, HALF), jnp.float32),
            pltpu.VMEM((ZROWS, HALF), jnp.float32),
            pltpu.VMEM_SHARED((N_NODES, HALF), jnp.float32),
        ],
    )(_sc_cnt_body)
    return main_call, cnt_call


def _fin_body(s_ref, cnt_ref, hr_ref, w2_ref, b2_ref, g_ref, be_ref, o_ref):
    cnt = cnt_ref[0][:, 0:1] + cnt_ref[1][:, 0:1]
    msum = (jnp.dot(s_ref[0], w2_ref[:HALF],
                    preferred_element_type=jnp.float32)
            + jnp.dot(s_ref[1], w2_ref[HALF:],
                      preferred_element_type=jnp.float32)
            + cnt * b2_ref[...])
    x = hr_ref[...] + msum / jnp.maximum(cnt, 1.0)
    mu = jnp.mean(x, axis=-1, keepdims=True)
    var = jnp.mean((x - mu) ** 2, axis=-1, keepdims=True)
    o_ref[...] = (x - mu) * lax.rsqrt(var + 1e-5) * g_ref[...] + be_ref[...]


def _fin(s2, cnt2, h_regional, w2, b2, gamma, beta):
    mblk = 2000
    return pl.pallas_call(
        _fin_body,
        out_shape=jax.ShapeDtypeStruct((N_NODES, NODE_DIM), jnp.float32),
        grid=(N_NODES // mblk,),
        in_specs=[pl.BlockSpec((2, mblk, HALF), lambda i: (0, i, 0)),
                  pl.BlockSpec((2, mblk, HALF), lambda i: (0, i, 0)),
                  pl.BlockSpec((mblk, NODE_DIM), lambda i: (i, 0)),
                  pl.BlockSpec((NODE_DIM, NODE_DIM), lambda i: (0, 0)),
                  pl.BlockSpec((1, NODE_DIM), lambda i: (0, 0)),
                  pl.BlockSpec((1, NODE_DIM), lambda i: (0, 0)),
                  pl.BlockSpec((1, NODE_DIM), lambda i: (0, 0))],
        out_specs=pl.BlockSpec((mblk, NODE_DIM), lambda i: (i, 0)),
    )(s2, cnt2, h_regional, w2, b2.reshape(1, NODE_DIM),
      gamma.reshape(1, NODE_DIM), beta.reshape(1, NODE_DIM))


def kernel(h_global, h_regional, cross_edge_index, cross_edge_attr,
           n_global, W1, b1, W2, b2, gamma, beta):
    senders = cross_edge_index[0, :E_HALF].astype(jnp.int32)
    receivers = cross_edge_index[1, :E_HALF].astype(jnp.int32)
    attr = cross_edge_attr[:E_HALF]

    tg = _mm(h_global, W1[:NODE_DIM], 2000).reshape(2 * N_NODES, HALF)
    tr = _mm(h_regional, W1[NODE_DIM:2 * NODE_DIM], 2000).reshape(
        2 * N_NODES, HALF)
    xe = _xe(attr, W1[2 * NODE_DIM:], b1).reshape(2 * E_HALF, HALF)

    snd = senders.reshape(N_CHUNKS, CHUNK // SUB, SUB)
    rcv = receivers.reshape(N_CHUNKS, CHUNK // SUB, SUB)

    main_call, cnt_call = _sc_calls()
    s2 = main_call(tg, tr, xe, snd, rcv)
    cnt2 = cnt_call(rcv)
    return _fin(s2.reshape(2, N_NODES, HALF), cnt2.reshape(2, N_NODES, HALF),
                h_regional, W2, b2, gamma, beta)


# speedup vs baseline: 3.3096x; 1.2342x over previous
import functools

import jax
import jax.numpy as jnp
from jax import lax
from jax.experimental import pallas as pl
from jax.experimental.pallas import tpu as pltpu
from jax.experimental.pallas import tpu_sc as plsc

N_NODES = 10000
NODE_DIM = 256
EDGE_DIM = 16
HALF = 128
E_HALF = 160000
CHUNK = 256
SUB = 64
N_CHUNKS = E_HALF // CHUNK
NS = 16
LANES = 16
ZROWS = 80
N_PIECES = N_NODES // ZROWS
N_PITER = (N_PIECES + NS - 1) // NS
N_ITER = (N_CHUNKS + NS - 1) // NS
NSUB = CHUNK // SUB


def _mm_body(x_ref, w_ref, o_ref):
    o_ref[0] = jnp.dot(x_ref[...], w_ref[...],
                       preferred_element_type=jnp.float32)


def _mm(x, w, mblk):
    m = x.shape[0]
    k = x.shape[1]
    return pl.pallas_call(
        _mm_body,
        out_shape=jax.ShapeDtypeStruct((2, m, HALF), jnp.float32),
        grid=(2, m // mblk),
        in_specs=[pl.BlockSpec((mblk, k), lambda j, i: (i, 0)),
                  pl.BlockSpec((k, HALF), lambda j, i: (0, j))],
        out_specs=pl.BlockSpec((1, mblk, HALF), lambda j, i: (j, i, 0)),
    )(x, w)


def _xe_body(a_ref, w_ref, b_ref, o_ref):
    o_ref[0] = (jnp.dot(a_ref[...], w_ref[...],
                        preferred_element_type=jnp.float32)
                + b_ref[pl.program_id(0)][None, :])


def _xe(attr, w1e, b1):
    mblk = 2000
    return pl.pallas_call(
        _xe_body,
        out_shape=jax.ShapeDtypeStruct((2, E_HALF, HALF), jnp.float32),
        grid=(2, E_HALF // mblk),
        in_specs=[pl.BlockSpec((mblk, EDGE_DIM), lambda j, i: (i, 0)),
                  pl.BlockSpec((EDGE_DIM, HALF), lambda j, i: (0, j)),
                  pl.BlockSpec((2, HALF), lambda j, i: (0, 0))],
        out_specs=pl.BlockSpec((1, mblk, HALF), lambda j, i: (j, i, 0)),
    )(attr, w1e, b1.reshape(2, HALF))


def _sc_main_body(tg, tr, xe, snd, rcv, zhbm, s2_out,
                  idx_s, idx_r, idx_rg, bg0, bg1, br0, br1, be, acc,
                  sem_g, sem_e, sem_s):
    bgs = (bg0, bg1)
    brs = (br0, br1)
    c = lax.axis_index("c")
    wid = lax.axis_index("s")
    coff = c * N_NODES

    def _zero(i, _):
        p = wid + i * NS

        @pl.when(p < N_PIECES)
        def _():
            r0 = p * ZROWS
            pltpu.sync_copy(zhbm.at[pl.ds(r0, ZROWS)],
                            acc.at[pl.ds(r0, ZROWS)])
        return 0
    lax.fori_loop(0, N_PITER, _zero, 0)
    plsc.subcore_barrier()

    def _chunk(g):
        base = g * CHUNK
        pltpu.sync_copy(snd.at[g], idx_s)
        pltpu.sync_copy(rcv.at[g], idx_r)
        for j in range(NSUB):
            for k in range(SUB // LANES):
                sl = pl.ds(k * LANES, LANES)
                idx_s[j, sl] = idx_s[j, sl] + coff
                idx_rg[j, sl] = idx_r[j, sl] + coff
        nd = [pltpu.async_copy(tg.at[idx_s.at[0]], bgs[0], sem_g),
              pltpu.async_copy(tr.at[idx_rg.at[0]], brs[0], sem_g)]
        sds = [None] * NSUB
        for j in range(NSUB):
            cur = j % 2
            bg, br = bgs[cur], brs[cur]
            ed = pltpu.async_copy(
                xe.at[pl.ds(c * E_HALF + base + j * SUB, SUB)], be, sem_e)
            for d in nd:
                d.wait()
            if j + 1 < NSUB:
                if j >= 1:
                    sds[j - 1].wait()
                nxt = 1 - cur
                nd = [pltpu.async_copy(tg.at[idx_s.at[j + 1]],
                                       bgs[nxt], sem_g),
                      pltpu.async_copy(tr.at[idx_rg.at[j + 1]],
                                       brs[nxt], sem_g)]
            ed.wait()

            def _edge(e, _):
                for k in range(HALF // LANES):
                    sl = pl.ds(k * LANES, LANES)
                    x = bg[e, sl] + br[e, sl] + be[e, sl]
                    bg[e, sl] = x / (1.0 + jnp.exp(-x))
                return 0
            lax.fori_loop(0, SUB, _edge, 0)

            sds[j] = pltpu.async_copy(bg, acc.at[idx_r.at[j]], sem_s,
                                      add=True)
        sds[NSUB - 2].wait()
        sds[NSUB - 1].wait()

    def _outer(i, _):
        g = wid + i * NS

        @pl.when(g < N_CHUNKS)
        def _():
            _chunk(g)
        return 0
    lax.fori_loop(0, N_ITER, _outer, 0)
    plsc.subcore_barrier()

    def _drain(i, _):
        p = wid + i * NS

        @pl.when(p < N_PIECES)
        def _():
            r0 = p * ZROWS
            pltpu.sync_copy(acc.at[pl.ds(r0, ZROWS)],
                            s2_out.at[pl.ds(c * N_NODES + r0, ZROWS)])
        return 0
    lax.fori_loop(0, N_PITER, _drain, 0)


def _sc_cnt_body(rcv, zhbm, cnt_out, idx_r, ones, cntacc):
    c = lax.axis_index("c")
    wid = lax.axis_index("s")
    w = c * NS + wid

    def _ofill(i, _):
        for k in range(HALF // LANES):
            ones[i, pl.ds(k * LANES, LANES)] = jnp.ones((LANES,),
                                                        jnp.float32)
        return 0
    lax.fori_loop(0, SUB, _ofill, 0)

    def _zero(i, _):
        p = wid + i * NS

        @pl.when(p < N_PIECES)
        def _():
            r0 = p * ZROWS
            pltpu.sync_copy(zhbm.at[pl.ds(r0, ZROWS)],
                            cntacc.at[pl.ds(r0, ZROWS)])
        return 0
    lax.fori_loop(0, N_PITER, _zero, 0)
    plsc.subcore_barrier()

    n_witer = (N_CHUNKS + 2 * NS - 1) // (2 * NS)
    def _outer(i, _):
        g = w + i * 2 * NS

        @pl.when(g < N_CHUNKS)
        def _():
            pltpu.sync_copy(rcv.at[g], idx_r)
            for j in range(CHUNK // SUB):
                pltpu.sync_copy(ones, cntacc.at[idx_r.at[j]], add=True)
        return 0
    lax.fori_loop(0, n_witer, _outer, 0)
    plsc.subcore_barrier()

    def _drain(i, _):
        p = wid + i * NS

        @pl.when(p < N_PIECES)
        def _():
            r0 = p * ZROWS
            pltpu.sync_copy(cntacc.at[pl.ds(r0, ZROWS)],
                            cnt_out.at[pl.ds(c * N_NODES + r0, ZROWS)])
        return 0
    lax.fori_loop(0, N_PITER, _drain, 0)


@functools.lru_cache(maxsize=None)
def _sc_calls():
    mesh = plsc.VectorSubcoreMesh(core_axis_name="c", subcore_axis_name="s")
    main_call = functools.partial(
        pl.kernel,
        out_type=jax.ShapeDtypeStruct((2 * N_NODES, HALF), jnp.float32),
        mesh=mesh,
        scratch_types=[
            pltpu.VMEM((CHUNK // SUB, SUB), jnp.int32),
            pltpu.VMEM((CHUNK // SUB, SUB), jnp.int32),
            pltpu.VMEM((CHUNK // SUB, SUB), jnp.int32),
            pltpu.VMEM((SUB, HALF), jnp.float32),
            pltpu.VMEM((SUB, HALF), jnp.float32),
            pltpu.VMEM((SUB, HALF), jnp.float32),
            pltpu.VMEM((SUB, HALF), jnp.float32),
            pltpu.VMEM((SUB, HALF), jnp.float32),
            pltpu.VMEM_SHARED((N_NODES, HALF), jnp.float32),
            pltpu.SemaphoreType.DMA,
            pltpu.SemaphoreType.DMA,
            pltpu.SemaphoreType.DMA,
        ],
    )(_sc_main_body)
    cnt_call = functools.partial(
        pl.kernel,
        out_type=jax.ShapeDtypeStruct((2 * N_NODES, HALF), jnp.float32),
        mesh=mesh,
        scratch_types=[
            pltpu.VMEM((CHUNK // SUB, SUB), jnp.int32),
            pltpu.VMEM((SUB, HALF), jnp.float32),
            pltpu.VMEM_SHARED((N_NODES, HALF), jnp.float32),
        ],
    )(_sc_cnt_body)
    return main_call, cnt_call


def _fin_body(s_ref, cnt_ref, hr_ref, w2_ref, b2_ref, g_ref, be_ref, o_ref):
    cnt = cnt_ref[0][:, 0:1] + cnt_ref[1][:, 0:1]
    msum = (jnp.dot(s_ref[0], w2_ref[:HALF],
                    preferred_element_type=jnp.float32)
            + jnp.dot(s_ref[1], w2_ref[HALF:],
                      preferred_element_type=jnp.float32)
            + cnt * b2_ref[...])
    x = hr_ref[...] + msum / jnp.maximum(cnt, 1.0)
    mu = jnp.mean(x, axis=-1, keepdims=True)
    var = jnp.mean((x - mu) ** 2, axis=-1, keepdims=True)
    o_ref[...] = (x - mu) * lax.rsqrt(var + 1e-5) * g_ref[...] + be_ref[...]


def _fin(s2, cnt2, h_regional, w2, b2, gamma, beta):
    mblk = 2000
    return pl.pallas_call(
        _fin_body,
        out_shape=jax.ShapeDtypeStruct((N_NODES, NODE_DIM), jnp.float32),
        grid=(N_NODES // mblk,),
        in_specs=[pl.BlockSpec((2, mblk, HALF), lambda i: (0, i, 0)),
                  pl.BlockSpec((2, mblk, HALF), lambda i: (0, i, 0)),
                  pl.BlockSpec((mblk, NODE_DIM), lambda i: (i, 0)),
                  pl.BlockSpec((NODE_DIM, NODE_DIM), lambda i: (0, 0)),
                  pl.BlockSpec((1, NODE_DIM), lambda i: (0, 0)),
                  pl.BlockSpec((1, NODE_DIM), lambda i: (0, 0)),
                  pl.BlockSpec((1, NODE_DIM), lambda i: (0, 0))],
        out_specs=pl.BlockSpec((mblk, NODE_DIM), lambda i: (i, 0)),
    )(s2, cnt2, h_regional, w2, b2.reshape(1, NODE_DIM),
      gamma.reshape(1, NODE_DIM), beta.reshape(1, NODE_DIM))


def kernel(h_global, h_regional, cross_edge_index, cross_edge_attr,
           n_global, W1, b1, W2, b2, gamma, beta):
    senders = cross_edge_index[0, :E_HALF].astype(jnp.int32)
    receivers = cross_edge_index[1, :E_HALF].astype(jnp.int32)
    attr = cross_edge_attr[:E_HALF]

    tg = _mm(h_global, W1[:NODE_DIM], 2000).reshape(2 * N_NODES, HALF)
    tr = _mm(h_regional, W1[NODE_DIM:2 * NODE_DIM], 2000).reshape(
        2 * N_NODES, HALF)
    xe = _xe(attr, W1[2 * NODE_DIM:], b1).reshape(2 * E_HALF, HALF)

    snd = senders.reshape(N_CHUNKS, CHUNK // SUB, SUB)
    rcv = receivers.reshape(N_CHUNKS, CHUNK // SUB, SUB)

    zhbm = jnp.zeros((N_NODES, HALF), jnp.float32)
    main_call, cnt_call = _sc_calls()
    s2 = main_call(tg, tr, xe, snd, rcv, zhbm)
    cnt2 = cnt_call(rcv, zhbm)
    return _fin(s2.reshape(2, N_NODES, HALF), cnt2.reshape(2, N_NODES, HALF),
                h_regional, W2, b2, gamma, beta)
